# baseline (speedup 1.0000x reference)
import jax
import jax.numpy as jnp
from jax import lax
from jax.experimental import pallas as pl
from jax.experimental.pallas import tpu as pltpu

B = 1024
D = 768
P = 64
POOL = 30
L = 3
K = 3

BS1 = 16
BS = 128


def _matvec_body(x_ref, w_ref, o_ref):
    x = x_ref[...].reshape(BS1 * D, P)
    v = lax.dot_general(w_ref[...], x, (((1,), (1,)), ((), ())),
                        preferred_element_type=jnp.float32)
    o_ref[...] = v[0]


def _rownorm(x):
    sq = x * x
    p = sq[:, 0:128]
    for c in range(1, 6):
        p = p + sq[:, c * 128:(c + 1) * 128]
    q = p[:, 0:8]
    for t in range(1, 16):
        q = q + p[:, 8 * t:8 * t + 8]
    a0 = q[:, 0:1] + q[:, 4:5]
    a2 = q[:, 2:3] + q[:, 6:7]
    a1 = q[:, 1:2] + q[:, 5:6]
    a3 = q[:, 3:4] + q[:, 7:8]
    tot = (a0 + a2) + (a1 + a3)
    return jnp.sqrt(tot)


def _main_body(s_ref, keys_ref, pool_ref, b_ref, out_ref, acc_ref):
    i = pl.program_id(0)

    @pl.when(i == 0)
    def _():
        acc_ref[0, 0] = 0.0

    s = s_ref[...] + b_ref[0, 0]
    s_norm = s / jnp.clip(_rownorm(s), 1e-12, None)

    keys = keys_ref[...]
    kn = keys / jnp.clip(_rownorm(keys), 1e-12, None)

    sim = lax.dot_general(s_norm, kn, (((1,), (1,)), ((), ())),
                          preferred_element_type=jnp.float32)

    iota = lax.broadcasted_iota(jnp.int32, (BS, POOL), 1)
    acc = 0.0
    simw = sim
    for k in range(K):
        m = jnp.max(simw, axis=1, keepdims=True)
        ismax = simw == m
        idxk = jnp.min(jnp.where(ismax, iota, POOL), axis=1, keepdims=True)
        sel = iota == idxk
        ohk = sel.astype(jnp.float32)
        for l in range(L):
            out_ref[:, k * L + l, :] = lax.dot_general(
                ohk, pool_ref[:, l, :], (((1,), (0,)), ((), ())),
                precision=lax.Precision.HIGHEST,
                preferred_element_type=jnp.float32)
        acc = acc + jnp.sum(m)
        simw = jnp.where(sel, -3e38, simw)

    acc_ref[0, 0] += acc


def kernel(summary, prompt_keys, prompt_values, W_map, b_map):
    b2 = b_map.reshape(1, 1)

    s_flat = pl.pallas_call(
        _matvec_body,
        grid=(B // BS1,),
        in_specs=[
            pl.BlockSpec((BS1, D, P), lambda i: (i, 0, 0)),
            pl.BlockSpec((1, P), lambda i: (0, 0)),
        ],
        out_specs=pl.BlockSpec((BS1 * D,), lambda i: (i,)),
        out_shape=jax.ShapeDtypeStruct((B * D,), jnp.float32),
    )(summary, W_map)

    s = s_flat.reshape(B, D)

    batched_prompt, out_acc = pl.pallas_call(
        _main_body,
        grid=(B // BS,),
        in_specs=[
            pl.BlockSpec((BS, D), lambda i: (i, 0)),
            pl.BlockSpec((POOL, D), lambda i: (0, 0)),
            pl.BlockSpec((POOL, L, D), lambda i: (0, 0, 0)),
            pl.BlockSpec(memory_space=pltpu.SMEM),
        ],
        out_specs=[
            pl.BlockSpec((BS, K * L, D), lambda i: (i, 0, 0)),
            pl.BlockSpec(memory_space=pltpu.SMEM),
        ],
        out_shape=[
            jax.ShapeDtypeStruct((B, K * L, D), jnp.float32),
            jax.ShapeDtypeStruct((1, 1), jnp.float32),
        ],
    )(s, prompt_keys, prompt_values, b2)
    reduce_sim = out_acc[0, 0] / B
    return (batched_prompt, reduce_sim)


# speedup vs baseline: 3.0053x; 3.0053x over previous
import jax
import jax.numpy as jnp
from jax import lax
from jax.experimental import pallas as pl
from jax.experimental.pallas import tpu as pltpu

B = 1024
D = 768
P = 64
POOL = 30
L = 3
K = 3

BS1 = 16
BS = 128


def _matvec_body(x_ref, w_ref, o_ref):
    x = x_ref[...].reshape(BS1 * P, D)
    w = w_ref[...]
    w_rep = jnp.concatenate([w] * BS1, axis=1)
    biota = lax.broadcasted_iota(jnp.int32, (BS1, BS1 * P), 0)
    kiota = lax.broadcasted_iota(jnp.int32, (BS1, BS1 * P), 1)
    lhs = jnp.where((kiota // P) == biota,
                    jnp.broadcast_to(w_rep, (BS1, BS1 * P)), 0.0)
    o_ref[...] = lax.dot_general(lhs, x, (((1,), (0,)), ((), ())),
                                 preferred_element_type=jnp.float32)


def _rownorm(x):
    sq = x * x
    p = sq[:, 0:128]
    for c in range(1, 6):
        p = p + sq[:, c * 128:(c + 1) * 128]
    q = p[:, 0:8]
    for t in range(1, 16):
        q = q + p[:, 8 * t:8 * t + 8]
    a0 = q[:, 0:1] + q[:, 4:5]
    a2 = q[:, 2:3] + q[:, 6:7]
    a1 = q[:, 1:2] + q[:, 5:6]
    a3 = q[:, 3:4] + q[:, 7:8]
    tot = (a0 + a2) + (a1 + a3)
    return jnp.sqrt(tot)


def _main_body(s_ref, keys_ref, pool_ref, b_ref, out_ref, acc_ref):
    i = pl.program_id(0)

    @pl.when(i == 0)
    def _():
        acc_ref[0, 0] = 0.0

    s = s_ref[...] + b_ref[0, 0]
    s_norm = s / jnp.clip(_rownorm(s), 1e-12, None)

    keys = keys_ref[...]
    kn = keys / jnp.clip(_rownorm(keys), 1e-12, None)

    sim = lax.dot_general(s_norm, kn, (((1,), (1,)), ((), ())),
                          preferred_element_type=jnp.float32)

    iota = lax.broadcasted_iota(jnp.int32, (BS, POOL), 1)
    acc = 0.0
    simw = sim
    for k in range(K):
        m = jnp.max(simw, axis=1, keepdims=True)
        ismax = simw == m
        idxk = jnp.min(jnp.where(ismax, iota, POOL), axis=1, keepdims=True)
        sel = iota == idxk
        ohk = sel.astype(jnp.float32)
        for l in range(L):
            out_ref[:, k * L + l, :] = lax.dot_general(
                ohk, pool_ref[:, l, :], (((1,), (0,)), ((), ())),
                precision=lax.Precision.HIGHEST,
                preferred_element_type=jnp.float32)
        acc = acc + jnp.sum(m)
        simw = jnp.where(sel, -3e38, simw)

    acc_ref[0, 0] += acc


def kernel(summary, prompt_keys, prompt_values, W_map, b_map):
    b2 = b_map.reshape(1, 1)
    summary_t = jnp.transpose(summary, (0, 2, 1))

    s = pl.pallas_call(
        _matvec_body,
        grid=(B // BS1,),
        in_specs=[
            pl.BlockSpec((BS1, P, D), lambda i: (i, 0, 0)),
            pl.BlockSpec((1, P), lambda i: (0, 0)),
        ],
        out_specs=pl.BlockSpec((BS1, D), lambda i: (i, 0)),
        out_shape=jax.ShapeDtypeStruct((B, D), jnp.float32),
    )(summary_t, W_map)

    batched_prompt, out_acc = pl.pallas_call(
        _main_body,
        grid=(B // BS,),
        in_specs=[
            pl.BlockSpec((BS, D), lambda i: (i, 0)),
            pl.BlockSpec((POOL, D), lambda i: (0, 0)),
            pl.BlockSpec((POOL, L, D), lambda i: (0, 0, 0)),
            pl.BlockSpec(memory_space=pltpu.SMEM),
        ],
        out_specs=[
            pl.BlockSpec((BS, K * L, D), lambda i: (i, 0, 0)),
            pl.BlockSpec(memory_space=pltpu.SMEM),
        ],
        out_shape=[
            jax.ShapeDtypeStruct((B, K * L, D), jnp.float32),
            jax.ShapeDtypeStruct((1, 1), jnp.float32),
        ],
    )(s, prompt_keys, prompt_values, b2)
    reduce_sim = out_acc[0, 0] / B
    return (batched_prompt, reduce_sim)


# speedup vs baseline: 4.0689x; 1.3539x over previous
import jax
import jax.numpy as jnp
from jax import lax
from jax.experimental import pallas as pl
from jax.experimental.pallas import tpu as pltpu

B = 1024
D = 768
P = 64
POOL = 30
L = 3
K = 3

BS1 = 16
BS = 128


def _matvec_body(x_ref, w_ref, o_ref):
    x = x_ref[...].reshape(BS1 * P, D)
    w = w_ref[...]
    w_rep = jnp.concatenate([w] * BS1, axis=1)
    biota = lax.broadcasted_iota(jnp.int32, (BS1, BS1 * P), 0)
    kiota = lax.broadcasted_iota(jnp.int32, (BS1, BS1 * P), 1)
    lhs = jnp.where((kiota // P) == biota,
                    jnp.broadcast_to(w_rep, (BS1, BS1 * P)), 0.0)
    o_ref[...] = lax.dot_general(lhs, x, (((1,), (0,)), ((), ())),
                                 preferred_element_type=jnp.float32)


def _rownorm(x):
    sq = x * x
    p = sq[:, 0:128]
    for c in range(1, 6):
        p = p + sq[:, c * 128:(c + 1) * 128]
    q = p[:, 0:8]
    for t in range(1, 16):
        q = q + p[:, 8 * t:8 * t + 8]
    a0 = q[:, 0:1] + q[:, 4:5]
    a2 = q[:, 2:3] + q[:, 6:7]
    a1 = q[:, 1:2] + q[:, 5:6]
    a3 = q[:, 3:4] + q[:, 7:8]
    tot = (a0 + a2) + (a1 + a3)
    return jnp.sqrt(tot)


def _main_body(s_ref, keys_ref, pool_ref, b_ref, out_ref, acc_ref):
    i = pl.program_id(0)

    @pl.when(i == 0)
    def _():
        acc_ref[0, 0] = 0.0

    s = s_ref[...] + b_ref[0, 0]
    s_norm = s / jnp.clip(_rownorm(s), 1e-12, None)

    keys = keys_ref[...]
    kn = keys / jnp.clip(_rownorm(keys), 1e-12, None)

    sim = lax.dot_general(s_norm, kn, (((1,), (1,)), ((), ())),
                          preferred_element_type=jnp.float32)

    iota = lax.broadcasted_iota(jnp.int32, (BS, POOL), 1)
    acc = 0.0
    simw = sim
    for k in range(K):
        m = jnp.max(simw, axis=1, keepdims=True)
        ismax = simw == m
        idxk = jnp.min(jnp.where(ismax, iota, POOL), axis=1, keepdims=True)
        sel = iota == idxk
        ohk = sel.astype(jnp.float32)
        for l in range(L):
            out_ref[k * L + l, :, :] = lax.dot_general(
                ohk, pool_ref[:, l, :], (((1,), (0,)), ((), ())),
                precision=lax.Precision.HIGHEST,
                preferred_element_type=jnp.float32)
        acc = acc + jnp.sum(m)
        simw = jnp.where(sel, -3e38, simw)

    acc_ref[0, 0] += acc


def kernel(summary, prompt_keys, prompt_values, W_map, b_map):
    b2 = b_map.reshape(1, 1)
    summary_t = jnp.transpose(summary, (0, 2, 1))

    s = pl.pallas_call(
        _matvec_body,
        grid=(B // BS1,),
        in_specs=[
            pl.BlockSpec((BS1, P, D), lambda i: (i, 0, 0)),
            pl.BlockSpec((1, P), lambda i: (0, 0)),
        ],
        out_specs=pl.BlockSpec((BS1, D), lambda i: (i, 0)),
        out_shape=jax.ShapeDtypeStruct((B, D), jnp.float32),
    )(summary_t, W_map)

    out_jmajor, out_acc = pl.pallas_call(
        _main_body,
        grid=(B // BS,),
        in_specs=[
            pl.BlockSpec((BS, D), lambda i: (i, 0)),
            pl.BlockSpec((POOL, D), lambda i: (0, 0)),
            pl.BlockSpec((POOL, L, D), lambda i: (0, 0, 0)),
            pl.BlockSpec(memory_space=pltpu.SMEM),
        ],
        out_specs=[
            pl.BlockSpec((K * L, BS, D), lambda i: (0, i, 0)),
            pl.BlockSpec(memory_space=pltpu.SMEM),
        ],
        out_shape=[
            jax.ShapeDtypeStruct((K * L, B, D), jnp.float32),
            jax.ShapeDtypeStruct((1, 1), jnp.float32),
        ],
    )(s, prompt_keys, prompt_values, b2)
    batched_prompt = jnp.transpose(out_jmajor, (1, 0, 2))
    reduce_sim = out_acc[0, 0] / B
    return (batched_prompt, reduce_sim)


# speedup vs baseline: 4.7365x; 1.1641x over previous
import jax
import jax.numpy as jnp
from jax import lax
from jax.experimental import pallas as pl
from jax.experimental.pallas import tpu as pltpu

B = 1024
D = 768
P = 64
POOL = 30
L = 3
K = 3

BS1 = 32
BS = 128


def _matvec_body(x_ref, w_ref, o_ref):
    x = x_ref[...].reshape(BS1 * P, D)
    w = w_ref[...]
    w_rep = jnp.concatenate([w] * BS1, axis=1)
    biota = lax.broadcasted_iota(jnp.int32, (BS1, BS1 * P), 0)
    kiota = lax.broadcasted_iota(jnp.int32, (BS1, BS1 * P), 1)
    lhs = jnp.where((kiota // P) == biota,
                    jnp.broadcast_to(w_rep, (BS1, BS1 * P)), 0.0)
    o_ref[...] = lax.dot_general(lhs, x, (((1,), (0,)), ((), ())),
                                 preferred_element_type=jnp.float32)


def _rownorm(x):
    sq = x * x
    p = sq[:, 0:128]
    for c in range(1, 6):
        p = p + sq[:, c * 128:(c + 1) * 128]
    q = p[:, 0:8]
    for t in range(1, 16):
        q = q + p[:, 8 * t:8 * t + 8]
    a0 = q[:, 0:1] + q[:, 4:5]
    a2 = q[:, 2:3] + q[:, 6:7]
    a1 = q[:, 1:2] + q[:, 5:6]
    a3 = q[:, 3:4] + q[:, 7:8]
    tot = (a0 + a2) + (a1 + a3)
    return jnp.sqrt(tot)


def _main_body(s_ref, keys_ref, pool_ref, b_ref, out_ref, acc_ref):
    i = pl.program_id(0)

    @pl.when(i == 0)
    def _():
        acc_ref[0, 0] = 0.0

    s = s_ref[...] + b_ref[0, 0]
    s_norm = s / jnp.clip(_rownorm(s), 1e-12, None)

    keys = keys_ref[...]
    kn = keys / jnp.clip(_rownorm(keys), 1e-12, None)

    sim = lax.dot_general(s_norm, kn, (((1,), (1,)), ((), ())),
                          preferred_element_type=jnp.float32)

    iota = lax.broadcasted_iota(jnp.int32, (BS, POOL), 1)
    acc = 0.0
    simw = sim
    for k in range(K):
        m = jnp.max(simw, axis=1, keepdims=True)
        ismax = simw == m
        idxk = jnp.min(jnp.where(ismax, iota, POOL), axis=1, keepdims=True)
        sel = iota == idxk
        ohk = sel.astype(jnp.float32)
        for l in range(L):
            out_ref[k * L + l, :, :] = lax.dot_general(
                ohk, pool_ref[:, l, :], (((1,), (0,)), ((), ())),
                precision=lax.Precision.HIGHEST,
                preferred_element_type=jnp.float32)
        acc = acc + jnp.sum(m)
        simw = jnp.where(sel, -3e38, simw)

    acc_ref[0, 0] += acc


def kernel(summary, prompt_keys, prompt_values, W_map, b_map):
    b2 = b_map.reshape(1, 1)
    summary_t = jnp.transpose(summary, (0, 2, 1))

    s = pl.pallas_call(
        _matvec_body,
        grid=(B // BS1,),
        in_specs=[
            pl.BlockSpec((BS1, P, D), lambda i: (i, 0, 0)),
            pl.BlockSpec((1, P), lambda i: (0, 0)),
        ],
        out_specs=pl.BlockSpec((BS1, D), lambda i: (i, 0)),
        out_shape=jax.ShapeDtypeStruct((B, D), jnp.float32),
    )(summary_t, W_map)

    out_jmajor, out_acc = pl.pallas_call(
        _main_body,
        grid=(B // BS,),
        in_specs=[
            pl.BlockSpec((BS, D), lambda i: (i, 0)),
            pl.BlockSpec((POOL, D), lambda i: (0, 0)),
            pl.BlockSpec((POOL, L, D), lambda i: (0, 0, 0)),
            pl.BlockSpec(memory_space=pltpu.SMEM),
        ],
        out_specs=[
            pl.BlockSpec((K * L, BS, D), lambda i: (0, i, 0)),
            pl.BlockSpec(memory_space=pltpu.SMEM),
        ],
        out_shape=[
            jax.ShapeDtypeStruct((K * L, B, D), jnp.float32),
            jax.ShapeDtypeStruct((1, 1), jnp.float32),
        ],
    )(s, prompt_keys, prompt_values, b2)
    batched_prompt = jnp.transpose(out_jmajor, (1, 0, 2))
    reduce_sim = out_acc[0, 0] / B
    return (batched_prompt, reduce_sim)
